# Initial kernel scaffold; baseline (speedup 1.0000x reference)
import jax
import jax.numpy as jnp
from jax.experimental import pallas as pl


def kernel(x, pos_table):
    raise NotImplementedError("write your pallas kernel here")


# speedup vs baseline: 1.7178x; 1.7178x over previous
import jax
import jax.numpy as jnp
from jax.experimental import pallas as pl

BLK = 256


def _add_kernel(x_ref, pos_ref, o_ref):
    o_ref[...] = x_ref[...] + pos_ref[...][None, :, :]


def kernel(x, pos_table):
    batch, ctx, emb = x.shape
    grid = (ctx // BLK,)
    return pl.pallas_call(
        _add_kernel,
        grid=grid,
        in_specs=[
            pl.BlockSpec((batch, BLK, emb), lambda i: (0, i, 0)),
            pl.BlockSpec((BLK, emb), lambda i: (i, 0)),
        ],
        out_specs=pl.BlockSpec((batch, BLK, emb), lambda i: (0, i, 0)),
        out_shape=jax.ShapeDtypeStruct(x.shape, x.dtype),
    )(x, pos_table)


# speedup vs baseline: 1.7226x; 1.0028x over previous
import jax
import jax.numpy as jnp
from jax.experimental import pallas as pl

BLK = 512


def _add_kernel(x_ref, pos_ref, o_ref):
    o_ref[...] = x_ref[...] + pos_ref[...][None, :, :]


def kernel(x, pos_table):
    batch, ctx, emb = x.shape
    grid = (ctx // BLK,)
    return pl.pallas_call(
        _add_kernel,
        grid=grid,
        in_specs=[
            pl.BlockSpec((batch, BLK, emb), lambda i: (0, i, 0)),
            pl.BlockSpec((BLK, emb), lambda i: (i, 0)),
        ],
        out_specs=pl.BlockSpec((batch, BLK, emb), lambda i: (0, i, 0)),
        out_shape=jax.ShapeDtypeStruct(x.shape, x.dtype),
    )(x, pos_table)
